# Initial kernel scaffold; baseline (speedup 1.0000x reference)
import jax
import jax.numpy as jnp
from jax.experimental import pallas as pl


def kernel(x, edge_index, Wl1, bl1, Wr1, br1, att1, bias1, Wl2, bl2, Wr2, br2, att2, bias2):
    raise NotImplementedError("write your pallas kernel here")


# speedup vs baseline: 10.0129x; 10.0129x over previous
import functools

import jax
import jax.numpy as jnp
from jax import lax
from jax.experimental import pallas as pl
from jax.experimental.pallas import tpu as pltpu
from jax.experimental.pallas import tpu_sc as plsc

N = 10000
NP = 10240
E_IN = 320000
E_REAL = E_IN + N
EP = 331776
CHUNK = 128
ACHUNK = 96
NEG_SLOPE = 0.2

ACC_ROWS = 10016

_GATHER_DNUMS = lax.GatherDimensionNumbers(
    offset_dims=(), collapsed_slice_dims=(0,), start_index_map=(0,))


def _lane_shuffle(v, ix):
    return lax.gather(v, ix[:, None], _GATHER_DNUMS, (1,),
                      mode=lax.GatherScatterMode.PROMISE_IN_BOUNDS)


def _mm_kernel(x_ref, wl_ref, wr_ref, bl_ref, br_ref,
               xla_ref, xlb_ref, xra_ref, xrb_ref):
    x = x_ref[...]
    f = xla_ref.shape[1]
    xl = jnp.dot(x, wl_ref[...], preferred_element_type=jnp.float32) + bl_ref[...]
    xr = jnp.dot(x, wr_ref[...], preferred_element_type=jnp.float32) + br_ref[...]
    xla_ref[...] = xl[:, :f]
    xlb_ref[...] = xl[:, f:]
    xra_ref[...] = xr[:, :f]
    xrb_ref[...] = xr[:, f:]


def _mm_halves(x, wl, wr, bl, br, fh):
    k = x.shape[1]
    f2 = 2 * fh
    rb = 1024
    out = jax.ShapeDtypeStruct((NP, fh), jnp.float32)
    return pl.pallas_call(
        _mm_kernel,
        grid=(NP // rb,),
        in_specs=[
            pl.BlockSpec((rb, k), lambda i: (i, 0)),
            pl.BlockSpec((k, f2), lambda i: (0, 0)),
            pl.BlockSpec((k, f2), lambda i: (0, 0)),
            pl.BlockSpec((1, f2), lambda i: (0, 0)),
            pl.BlockSpec((1, f2), lambda i: (0, 0)),
        ],
        out_specs=[pl.BlockSpec((rb, fh), lambda i: (i, 0))] * 4,
        out_shape=[out] * 4,
    )(x, wl, wr, bl.reshape(1, -1), br.reshape(1, -1))


def _mm2_kernel(ha_ref, hb_ref, wl_ref, wr_ref, bl_ref, br_ref,
                xla_ref, xlb_ref, xra_ref, xrb_ref):
    ha = ha_ref[...]
    hb = hb_ref[...]
    f = xla_ref.shape[1]
    xl = (jnp.dot(ha, wl_ref[:128, :], preferred_element_type=jnp.float32)
          + jnp.dot(hb, wl_ref[128:, :], preferred_element_type=jnp.float32)
          + bl_ref[...])
    xr = (jnp.dot(ha, wr_ref[:128, :], preferred_element_type=jnp.float32)
          + jnp.dot(hb, wr_ref[128:, :], preferred_element_type=jnp.float32)
          + br_ref[...])
    xla_ref[...] = xl[:, :f]
    xlb_ref[...] = xl[:, f:]
    xra_ref[...] = xr[:, :f]
    xrb_ref[...] = xr[:, f:]


def _mm2_halves(ha, hb, wl, wr, bl, br, fh):
    f2 = 2 * fh
    rb = 1024
    out = jax.ShapeDtypeStruct((NP, fh), jnp.float32)
    return pl.pallas_call(
        _mm2_kernel,
        grid=(NP // rb,),
        in_specs=[
            pl.BlockSpec((rb, 128), lambda i: (i, 0)),
            pl.BlockSpec((rb, 128), lambda i: (i, 0)),
            pl.BlockSpec((256, f2), lambda i: (0, 0)),
            pl.BlockSpec((256, f2), lambda i: (0, 0)),
            pl.BlockSpec((1, f2), lambda i: (0, 0)),
            pl.BlockSpec((1, f2), lambda i: (0, 0)),
        ],
        out_specs=[pl.BlockSpec((rb, fh), lambda i: (i, 0))] * 4,
        out_shape=[out] * 4,
    )(ha, hb, wl, wr, bl.reshape(1, -1), br.reshape(1, -1))


def _make_score_kernel(f, h):
    fh = f // 2
    c_per_h = f // h
    per_w = EP // 32
    n_chunks = per_w // CHUNK
    mesh = plsc.VectorSubcoreMesh(core_axis_name="c", subcore_axis_name="s")

    @functools.partial(
        pl.kernel,
        out_type=jax.ShapeDtypeStruct((h, EP), jnp.float32),
        mesh=mesh,
        compiler_params=pltpu.CompilerParams(use_tc_tiling_on_sc=False),
        scratch_types=[
            pltpu.VMEM((CHUNK,), jnp.int32),
            pltpu.VMEM((CHUNK,), jnp.int32),
            pltpu.VMEM((CHUNK, fh), jnp.float32),
            pltpu.VMEM((CHUNK, fh), jnp.float32),
            pltpu.VMEM((CHUNK, fh), jnp.float32),
            pltpu.VMEM((CHUNK, fh), jnp.float32),
            pltpu.VMEM((f,), jnp.float32),
            pltpu.VMEM((h, CHUNK), jnp.float32),
            pltpu.SemaphoreType.DMA,
            pltpu.SemaphoreType.DMA,
            pltpu.SemaphoreType.DMA,
            pltpu.SemaphoreType.DMA,
        ],
    )
    def score_k(xla, xlb, xra, xrb, src, dst, att, scores,
                idx_s, idx_d, rla, rlb, rra, rrb,
                att_v, sc_buf, sem0, sem1, sem2, sem3):
        cid = lax.axis_index("c")
        sid = lax.axis_index("s")
        wid = sid * 2 + cid
        base = wid * per_w
        pltpu.sync_copy(att, att_v)
        iota = lax.iota(jnp.int32, 16)
        bfly = [iota ^ k for k in (8, 4, 2, 1)]

        def chunk_body(ci, _):
            start = base + ci * CHUNK
            pltpu.sync_copy(src.at[pl.ds(start, CHUNK)], idx_s)
            pltpu.sync_copy(dst.at[pl.ds(start, CHUNK)], idx_d)
            cp0 = pltpu.async_copy(xla.at[idx_s], rla, sem0)
            cp1 = pltpu.async_copy(xlb.at[idx_s], rlb, sem1)
            cp2 = pltpu.async_copy(xra.at[idx_d], rra, sem2)
            cp3 = pltpu.async_copy(xrb.at[idx_d], rrb, sem3)
            cp0.wait()
            cp1.wait()
            cp2.wait()
            cp3.wait()

            def grp_body(g, _):
                vecs = [jnp.zeros((16,), jnp.float32) for _ in range(h)]
                for j16 in range(16):
                    j = g * 16 + j16
                    for hh in range(h):
                        acc = jnp.zeros((16,), jnp.float32)
                        for qq in range(c_per_h // 16):
                            gg = hh * c_per_h + qq * 16
                            if gg < fh:
                                vl = rla[j, pl.ds(gg, 16)]
                                vr = rra[j, pl.ds(gg, 16)]
                            else:
                                vl = rlb[j, pl.ds(gg - fh, 16)]
                                vr = rrb[j, pl.ds(gg - fh, 16)]
                            v = vl + vr
                            lr = jnp.maximum(v, NEG_SLOPE * v)
                            acc = acc + lr * att_v[pl.ds(gg, 16)]
                        for ix in bfly:
                            acc = acc + _lane_shuffle(acc, ix)
                        vecs[hh] = jnp.where(iota == j16, acc, vecs[hh])
                for hh in range(h):
                    sc_buf[hh, pl.ds(g * 16, 16)] = vecs[hh]
                return 0

            lax.fori_loop(0, CHUNK // 16, grp_body, 0)
            pltpu.sync_copy(sc_buf, scores.at[:, pl.ds(start, CHUNK)])
            return 0

        lax.fori_loop(0, n_chunks, chunk_body, 0)

    return score_k


def _make_agg_kernel(f, h, relu):
    fh = f // 2
    heads_own = max(h // 2, 1)
    sl_per_head = (fh // 16) // heads_own
    w = fh + 16
    per_t = EP // 16
    n_chunks = per_t // ACHUNK
    mesh = plsc.VectorSubcoreMesh(core_axis_name="c", subcore_axis_name="s")
    out = jax.ShapeDtypeStruct((NP, fh), jnp.float32)

    @functools.partial(
        pl.kernel,
        out_type=[out, out],
        mesh=mesh,
        compiler_params=pltpu.CompilerParams(use_tc_tiling_on_sc=False),
        scratch_types=[
            pltpu.VMEM((ACHUNK,), jnp.int32),
            pltpu.VMEM((ACHUNK,), jnp.int32),
            pltpu.VMEM((ACHUNK, fh), jnp.float32),
            pltpu.VMEM((h, ACHUNK), jnp.float32),
            pltpu.VMEM((ACHUNK, w), jnp.float32),
            pltpu.VMEM((16, w), jnp.float32),
            pltpu.VMEM((16, w), jnp.float32),
            pltpu.VMEM((16, fh), jnp.float32),
            pltpu.VMEM((fh,), jnp.float32),
            pltpu.VMEM_SHARED((ACC_ROWS, w), jnp.float32),
            pltpu.SemaphoreType.DMA,
        ],
    )
    def agg_k(xla, xlb, src, dst, scores, bias, out_a, out_b,
              idx_s, idx_d, rows, sc_chunk, stage, zbuf, nblk, oblk,
              bias_v, acc, sem0):
        cid = lax.axis_index("c")
        sid = lax.axis_index("s")
        pltpu.sync_copy(bias.at[pl.ds(cid * fh, fh)], bias_v)

        zero = jnp.zeros((16,), jnp.float32)
        for r in range(16):
            for q in range(w // 16):
                zbuf[r, pl.ds(q * 16, 16)] = zero
        zblocks = ACC_ROWS // 16

        def zero_body(k, _):
            b = sid + 16 * k

            @pl.when(b < zblocks)
            def _():
                pltpu.sync_copy(zbuf, acc.at[pl.ds(b * 16, 16), :])

            return 0

        lax.fori_loop(0, (zblocks + 15) // 16, zero_body, 0)
        plsc.subcore_barrier()

        base = sid * per_t
        col0 = cid * (h // 2)
        iota = lax.iota(jnp.int32, 16)

        def chunk_body(ci, _):
            start = base + ci * ACHUNK
            pltpu.sync_copy(src.at[pl.ds(start, ACHUNK)], idx_s)
            pltpu.sync_copy(dst.at[pl.ds(start, ACHUNK)], idx_d)

            @pl.when(cid == 0)
            def _():
                pltpu.async_copy(xla.at[idx_s], rows, sem0).wait()

            @pl.when(cid == 1)
            def _():
                pltpu.async_copy(xlb.at[idx_s], rows, sem0).wait()

            pltpu.sync_copy(scores.at[:, pl.ds(start, ACHUNK)], sc_chunk)

            def grp_body(g, _):
                ev_grp = []
                for hl in range(heads_own):
                    sv = sc_chunk[col0 + hl, pl.ds(g * 16, 16)]
                    ev_grp.append(jnp.exp(sv))
                for j16 in range(16):
                    j = g * 16 + j16
                    evecs = [jnp.full((16,), ev_grp[hl][j16], jnp.float32)
                             for hl in range(heads_own)]
                    for q in range(fh // 16):
                        hl = q // sl_per_head
                        sl = pl.ds(q * 16, 16)
                        stage[j, sl] = rows[j, sl] * evecs[hl]
                    tail = jnp.zeros((16,), jnp.float32)
                    for hl in range(heads_own):
                        tail = jnp.where(iota == hl, evecs[hl], tail)
                    stage[j, pl.ds(fh, 16)] = tail
                return 0

            lax.fori_loop(0, ACHUNK // 16, grp_body, 0)
            pltpu.sync_copy(stage, acc.at[idx_d], add=True)
            return 0

        lax.fori_loop(0, n_chunks, chunk_body, 0)
        plsc.subcore_barrier()

        n_blocks = N // 16

        def norm_body(k, _):
            b = sid + 16 * k

            @pl.when(b < n_blocks)
            def _():
                r0 = b * 16
                pltpu.sync_copy(acc.at[pl.ds(r0, 16), :], nblk)
                for r in range(16):
                    dv = nblk[r, pl.ds(fh, 16)]
                    invs = []
                    for hl in range(heads_own):
                        invs.append(1.0 / (jnp.full((16,), dv[hl], jnp.float32) + 1e-16))
                    for q in range(fh // 16):
                        hl = q // sl_per_head
                        sl = pl.ds(q * 16, 16)
                        o = nblk[r, sl] * invs[hl] + bias_v[sl]
                        if relu:
                            o = jnp.maximum(o, 0.0)
                        oblk[r, sl] = o

                @pl.when(cid == 0)
                def _():
                    pltpu.sync_copy(oblk, out_a.at[pl.ds(r0, 16), :])

                @pl.when(cid == 1)
                def _():
                    pltpu.sync_copy(oblk, out_b.at[pl.ds(r0, 16), :])

            return 0

        lax.fori_loop(0, (n_blocks + 15) // 16, norm_body, 0)

    return agg_k


_score1 = _make_score_kernel(256, 4)
_agg1 = _make_agg_kernel(256, 4, True)
_score2 = _make_score_kernel(128, 1)
_agg2 = _make_agg_kernel(128, 1, False)


def kernel(x, edge_index, Wl1, bl1, Wr1, br1, att1, bias1,
           Wl2, bl2, Wr2, br2, att2, bias2):
    loop = jnp.arange(N, dtype=jnp.int32)
    padn = jnp.full((EP - E_REAL,), N, dtype=jnp.int32)
    src = jnp.concatenate([edge_index[0].astype(jnp.int32), loop, padn])
    dst = jnp.concatenate([edge_index[1].astype(jnp.int32), loop, padn])

    x_pad = jnp.zeros((NP, x.shape[1]), jnp.float32).at[:N].set(x)

    xla1, xlb1, xra1, xrb1 = _mm_halves(x_pad, Wl1, Wr1, bl1, br1, 128)
    s1 = _score1(xla1, xlb1, xra1, xrb1, src, dst, att1.reshape(-1))
    ha, hb = _agg1(xla1, xlb1, src, dst, s1, bias1)

    xla2, xlb2, xra2, xrb2 = _mm2_halves(ha, hb, Wl2, Wr2, bl2, br2, 64)
    s2 = _score2(xla2, xlb2, xra2, xrb2, src, dst, att2.reshape(-1))
    oa, ob = _agg2(xla2, xlb2, src, dst, s2, bias2)

    return jnp.concatenate([oa[:N], ob[:N]], axis=1)


# speedup vs baseline: 12.7554x; 1.2739x over previous
import functools

import jax
import jax.numpy as jnp
from jax import lax
from jax.experimental import pallas as pl
from jax.experimental.pallas import tpu as pltpu
from jax.experimental.pallas import tpu_sc as plsc

N = 10000
NP = 10240
E_IN = 320000
E_REAL = E_IN + N
EP = 331776
CHUNK = 96
ACHUNK = 96
NEG_SLOPE = 0.2

ACC_ROWS = 10016

_GATHER_DNUMS = lax.GatherDimensionNumbers(
    offset_dims=(), collapsed_slice_dims=(0,), start_index_map=(0,))


def _lane_shuffle(v, ix):
    return lax.gather(v, ix[:, None], _GATHER_DNUMS, (1,),
                      mode=lax.GatherScatterMode.PROMISE_IN_BOUNDS)


def _mm_kernel(x_ref, wl_ref, wr_ref, bl_ref, br_ref,
               xla_ref, xlb_ref, xra_ref, xrb_ref):
    x = x_ref[...]
    f = xla_ref.shape[1]
    xl = jnp.dot(x, wl_ref[...], preferred_element_type=jnp.float32) + bl_ref[...]
    xr = jnp.dot(x, wr_ref[...], preferred_element_type=jnp.float32) + br_ref[...]
    xla_ref[...] = xl[:, :f]
    xlb_ref[...] = xl[:, f:]
    xra_ref[...] = xr[:, :f]
    xrb_ref[...] = xr[:, f:]


def _mm_halves(x, wl, wr, bl, br, fh):
    k = x.shape[1]
    f2 = 2 * fh
    rb = 1024
    out = jax.ShapeDtypeStruct((NP, fh), jnp.float32)
    return pl.pallas_call(
        _mm_kernel,
        grid=(NP // rb,),
        in_specs=[
            pl.BlockSpec((rb, k), lambda i: (i, 0)),
            pl.BlockSpec((k, f2), lambda i: (0, 0)),
            pl.BlockSpec((k, f2), lambda i: (0, 0)),
            pl.BlockSpec((1, f2), lambda i: (0, 0)),
            pl.BlockSpec((1, f2), lambda i: (0, 0)),
        ],
        out_specs=[pl.BlockSpec((rb, fh), lambda i: (i, 0))] * 4,
        out_shape=[out] * 4,
    )(x, wl, wr, bl.reshape(1, -1), br.reshape(1, -1))


def _mm2_kernel(ha_ref, hb_ref, wl_ref, wr_ref, bl_ref, br_ref,
                xla_ref, xlb_ref, xra_ref, xrb_ref):
    ha = ha_ref[...]
    hb = hb_ref[...]
    f = xla_ref.shape[1]
    xl = (jnp.dot(ha, wl_ref[:128, :], preferred_element_type=jnp.float32)
          + jnp.dot(hb, wl_ref[128:, :], preferred_element_type=jnp.float32)
          + bl_ref[...])
    xr = (jnp.dot(ha, wr_ref[:128, :], preferred_element_type=jnp.float32)
          + jnp.dot(hb, wr_ref[128:, :], preferred_element_type=jnp.float32)
          + br_ref[...])
    xla_ref[...] = xl[:, :f]
    xlb_ref[...] = xl[:, f:]
    xra_ref[...] = xr[:, :f]
    xrb_ref[...] = xr[:, f:]


def _mm2_halves(ha, hb, wl, wr, bl, br, fh):
    f2 = 2 * fh
    rb = 1024
    out = jax.ShapeDtypeStruct((NP, fh), jnp.float32)
    return pl.pallas_call(
        _mm2_kernel,
        grid=(NP // rb,),
        in_specs=[
            pl.BlockSpec((rb, 128), lambda i: (i, 0)),
            pl.BlockSpec((rb, 128), lambda i: (i, 0)),
            pl.BlockSpec((256, f2), lambda i: (0, 0)),
            pl.BlockSpec((256, f2), lambda i: (0, 0)),
            pl.BlockSpec((1, f2), lambda i: (0, 0)),
            pl.BlockSpec((1, f2), lambda i: (0, 0)),
        ],
        out_specs=[pl.BlockSpec((rb, fh), lambda i: (i, 0))] * 4,
        out_shape=[out] * 4,
    )(ha, hb, wl, wr, bl.reshape(1, -1), br.reshape(1, -1))


def _make_score_kernel(f, h):
    fh = f // 2
    c_per_h = f // h
    per_w = EP // 32
    n_chunks = per_w // CHUNK
    mesh = plsc.VectorSubcoreMesh(core_axis_name="c", subcore_axis_name="s")

    row_t = pltpu.VMEM((CHUNK, fh), jnp.float32)

    @functools.partial(
        pl.kernel,
        out_type=jax.ShapeDtypeStruct((h, EP), jnp.float32),
        mesh=mesh,
        compiler_params=pltpu.CompilerParams(use_tc_tiling_on_sc=False),
        scratch_types=[
            pltpu.VMEM((CHUNK,), jnp.int32),
            pltpu.VMEM((CHUNK,), jnp.int32),
            pltpu.VMEM((CHUNK,), jnp.int32),
            pltpu.VMEM((CHUNK,), jnp.int32),
            row_t, row_t, row_t, row_t,
            row_t, row_t, row_t, row_t,
            pltpu.VMEM((f,), jnp.float32),
            pltpu.VMEM((h, CHUNK), jnp.float32),
            pltpu.SemaphoreType.DMA,
            pltpu.SemaphoreType.DMA,
        ],
    )
    def score_k(xla, xlb, xra, xrb, src, dst, att, scores,
                is0, id0, is1, id1,
                rla0, rlb0, rra0, rrb0, rla1, rlb1, rra1, rrb1,
                att_v, sc_buf, sem0, sem1):
        cid = lax.axis_index("c")
        sid = lax.axis_index("s")
        wid = sid * 2 + cid
        base = wid * per_w
        pltpu.sync_copy(att, att_v)
        iota = lax.iota(jnp.int32, 16)
        bfly = [iota ^ k for k in (8, 4, 2, 1)]
        bufs = ((is0, id0, rla0, rlb0, rra0, rrb0, sem0),
                (is1, id1, rla1, rlb1, rra1, rrb1, sem1))

        def fetch(ci, buf):
            idx_s, idx_d, rla, rlb, rra, rrb, sem = buf
            start = base + ci * CHUNK
            pltpu.sync_copy(src.at[pl.ds(start, CHUNK)], idx_s)
            pltpu.sync_copy(dst.at[pl.ds(start, CHUNK)], idx_d)
            pltpu.async_copy(xla.at[idx_s], rla, sem)
            pltpu.async_copy(xlb.at[idx_s], rlb, sem)
            pltpu.async_copy(xra.at[idx_d], rra, sem)
            pltpu.async_copy(xrb.at[idx_d], rrb, sem)

        def sweep(ci, buf):
            idx_s, idx_d, rla, rlb, rra, rrb, sem = buf
            start = base + ci * CHUNK
            pltpu.make_async_copy(xla.at[idx_s], rla, sem).wait()
            pltpu.make_async_copy(xlb.at[idx_s], rlb, sem).wait()
            pltpu.make_async_copy(xra.at[idx_d], rra, sem).wait()
            pltpu.make_async_copy(xrb.at[idx_d], rrb, sem).wait()

            def grp_body(g, _):
                vecs = [jnp.zeros((16,), jnp.float32) for _ in range(h)]
                for j16 in range(16):
                    j = g * 16 + j16
                    for hh in range(h):
                        acc = jnp.zeros((16,), jnp.float32)
                        for qq in range(c_per_h // 16):
                            gg = hh * c_per_h + qq * 16
                            if gg < fh:
                                vl = rla[j, pl.ds(gg, 16)]
                                vr = rra[j, pl.ds(gg, 16)]
                            else:
                                vl = rlb[j, pl.ds(gg - fh, 16)]
                                vr = rrb[j, pl.ds(gg - fh, 16)]
                            v = vl + vr
                            lr = jnp.maximum(v, NEG_SLOPE * v)
                            acc = acc + lr * att_v[pl.ds(gg, 16)]
                        for ix in bfly:
                            acc = acc + _lane_shuffle(acc, ix)
                        vecs[hh] = jnp.where(iota == j16, acc, vecs[hh])
                for hh in range(h):
                    sc_buf[hh, pl.ds(g * 16, 16)] = vecs[hh]
                return 0

            lax.fori_loop(0, CHUNK // 16, grp_body, 0)

            @pl.when(ci + 2 < n_chunks)
            def _():
                fetch(ci + 2, buf)

            pltpu.sync_copy(sc_buf, scores.at[:, pl.ds(start, CHUNK)])

        fetch(0, bufs[0])
        fetch(1, bufs[1])

        def pair_body(g, _):
            sweep(2 * g, bufs[0])
            sweep(2 * g + 1, bufs[1])
            return 0

        lax.fori_loop(0, n_chunks // 2, pair_body, 0)

    return score_k


def _make_agg_kernel(f, h, relu, achunk):
    fh = f // 2
    heads_own = max(h // 2, 1)
    sl_per_head = (fh // 16) // heads_own
    w = fh + 16
    per_t = EP // 16
    n_chunks = per_t // achunk
    mesh = plsc.VectorSubcoreMesh(core_axis_name="c", subcore_axis_name="s")
    out = jax.ShapeDtypeStruct((NP, fh), jnp.float32)

    @functools.partial(
        pl.kernel,
        out_type=[out, out],
        mesh=mesh,
        compiler_params=pltpu.CompilerParams(use_tc_tiling_on_sc=False),
        scratch_types=[
            pltpu.VMEM((achunk,), jnp.int32),
            pltpu.VMEM((achunk,), jnp.int32),
            pltpu.VMEM((achunk,), jnp.int32),
            pltpu.VMEM((achunk, fh), jnp.float32),
            pltpu.VMEM((achunk, fh), jnp.float32),
            pltpu.VMEM((h, achunk), jnp.float32),
            pltpu.VMEM((achunk, w), jnp.float32),
            pltpu.VMEM((16, w), jnp.float32),
            pltpu.VMEM((16, w), jnp.float32),
            pltpu.VMEM((16, fh), jnp.float32),
            pltpu.VMEM((fh,), jnp.float32),
            pltpu.VMEM_SHARED((ACC_ROWS, w), jnp.float32),
            pltpu.SemaphoreType.DMA,
            pltpu.SemaphoreType.DMA,
        ],
    )
    def agg_k(xla, xlb, src, dst, scores, bias, out_a, out_b,
              is0, is1, idx_d, rows0, rows1, sc_chunk, stage, zbuf, nblk, oblk,
              bias_v, acc, sem0, sem1):
        cid = lax.axis_index("c")
        sid = lax.axis_index("s")
        pltpu.sync_copy(bias.at[pl.ds(cid * fh, fh)], bias_v)

        zero = jnp.zeros((16,), jnp.float32)
        for r in range(16):
            for q in range(w // 16):
                zbuf[r, pl.ds(q * 16, 16)] = zero
        zblocks = ACC_ROWS // 16

        def zero_body(k, _):
            b = sid + 16 * k

            @pl.when(b < zblocks)
            def _():
                pltpu.sync_copy(zbuf, acc.at[pl.ds(b * 16, 16), :])

            return 0

        lax.fori_loop(0, (zblocks + 15) // 16, zero_body, 0)
        plsc.subcore_barrier()

        base = sid * per_t
        col0 = cid * (h // 2)
        iota = lax.iota(jnp.int32, 16)
        bufs = ((is0, rows0, sem0), (is1, rows1, sem1))

        def fetch(ci, buf):
            idx_s, rows, sem = buf
            start = base + ci * achunk
            pltpu.sync_copy(src.at[pl.ds(start, achunk)], idx_s)

            @pl.when(cid == 0)
            def _():
                pltpu.async_copy(xla.at[idx_s], rows, sem)

            @pl.when(cid == 1)
            def _():
                pltpu.async_copy(xlb.at[idx_s], rows, sem)

        def sweep(ci, buf):
            idx_s, rows, sem = buf
            start = base + ci * achunk
            pltpu.make_async_copy(xla.at[idx_s], rows, sem).wait()
            pltpu.sync_copy(dst.at[pl.ds(start, achunk)], idx_d)
            pltpu.sync_copy(scores.at[:, pl.ds(start, achunk)], sc_chunk)

            def grp_body(g, _):
                ev_grp = []
                for hl in range(heads_own):
                    sv = sc_chunk[col0 + hl, pl.ds(g * 16, 16)]
                    ev_grp.append(jnp.exp(sv))
                for j16 in range(16):
                    j = g * 16 + j16
                    evecs = [jnp.full((16,), ev_grp[hl][j16], jnp.float32)
                             for hl in range(heads_own)]
                    for q in range(fh // 16):
                        hl = q // sl_per_head
                        sl = pl.ds(q * 16, 16)
                        stage[j, sl] = rows[j, sl] * evecs[hl]
                    tail = jnp.zeros((16,), jnp.float32)
                    for hl in range(heads_own):
                        tail = jnp.where(iota == hl, evecs[hl], tail)
                    stage[j, pl.ds(fh, 16)] = tail
                return 0

            lax.fori_loop(0, achunk // 16, grp_body, 0)

            @pl.when(ci + 2 < n_chunks)
            def _():
                fetch(ci + 2, buf)

            pltpu.sync_copy(stage, acc.at[idx_d], add=True)

        fetch(0, bufs[0])
        fetch(1, bufs[1])

        def pair_body(g, _):
            sweep(2 * g, bufs[0])
            sweep(2 * g + 1, bufs[1])
            return 0

        lax.fori_loop(0, n_chunks // 2, pair_body, 0)
        plsc.subcore_barrier()

        n_blocks = N // 16

        def norm_body(k, _):
            b = sid + 16 * k

            @pl.when(b < n_blocks)
            def _():
                r0 = b * 16
                pltpu.sync_copy(acc.at[pl.ds(r0, 16), :], nblk)
                for r in range(16):
                    dv = nblk[r, pl.ds(fh, 16)]
                    invs = []
                    for hl in range(heads_own):
                        invs.append(1.0 / (jnp.full((16,), dv[hl], jnp.float32) + 1e-16))
                    for q in range(fh // 16):
                        hl = q // sl_per_head
                        sl = pl.ds(q * 16, 16)
                        o = nblk[r, sl] * invs[hl] + bias_v[sl]
                        if relu:
                            o = jnp.maximum(o, 0.0)
                        oblk[r, sl] = o

                @pl.when(cid == 0)
                def _():
                    pltpu.sync_copy(oblk, out_a.at[pl.ds(r0, 16), :])

                @pl.when(cid == 1)
                def _():
                    pltpu.sync_copy(oblk, out_b.at[pl.ds(r0, 16), :])

            return 0

        lax.fori_loop(0, (n_blocks + 15) // 16, norm_body, 0)

    return agg_k


_score1 = _make_score_kernel(256, 4)
_agg1 = _make_agg_kernel(256, 4, True, 64)
_score2 = _make_score_kernel(128, 1)
_agg2 = _make_agg_kernel(128, 1, False, 96)


def kernel(x, edge_index, Wl1, bl1, Wr1, br1, att1, bias1,
           Wl2, bl2, Wr2, br2, att2, bias2):
    loop = jnp.arange(N, dtype=jnp.int32)
    padn = jnp.full((EP - E_REAL,), N, dtype=jnp.int32)
    src = jnp.concatenate([edge_index[0].astype(jnp.int32), loop, padn])
    dst = jnp.concatenate([edge_index[1].astype(jnp.int32), loop, padn])

    x_pad = jnp.zeros((NP, x.shape[1]), jnp.float32).at[:N].set(x)

    xla1, xlb1, xra1, xrb1 = _mm_halves(x_pad, Wl1, Wr1, bl1, br1, 128)
    s1 = _score1(xla1, xlb1, xra1, xrb1, src, dst, att1.reshape(-1))
    ha, hb = _agg1(xla1, xlb1, src, dst, s1, bias1)

    xla2, xlb2, xra2, xrb2 = _mm2_halves(ha, hb, Wl2, Wr2, bl2, br2, 64)
    s2 = _score2(xla2, xlb2, xra2, xrb2, src, dst, att2.reshape(-1))
    oa, ob = _agg2(xla2, xlb2, src, dst, s2, bias2)

    return jnp.concatenate([oa[:N], ob[:N]], axis=1)


# speedup vs baseline: 13.2222x; 1.0366x over previous
import functools

import jax
import jax.numpy as jnp
from jax import lax
from jax.experimental import pallas as pl
from jax.experimental.pallas import tpu as pltpu
from jax.experimental.pallas import tpu_sc as plsc

N = 10000
NP = 10240
E_IN = 320000
E_REAL = E_IN + N
EP = 331776
CHUNK = 96
ACHUNK = 96
NEG_SLOPE = 0.2

ACC_ROWS = 10016

_GATHER_DNUMS = lax.GatherDimensionNumbers(
    offset_dims=(), collapsed_slice_dims=(0,), start_index_map=(0,))


def _lane_shuffle(v, ix):
    return lax.gather(v, ix[:, None], _GATHER_DNUMS, (1,),
                      mode=lax.GatherScatterMode.PROMISE_IN_BOUNDS)


def _mm_kernel(x_ref, wl_ref, wr_ref, bl_ref, br_ref,
               xla_ref, xlb_ref, xra_ref, xrb_ref):
    x = x_ref[...]
    f = xla_ref.shape[1]
    xl = jnp.dot(x, wl_ref[...], preferred_element_type=jnp.float32) + bl_ref[...]
    xr = jnp.dot(x, wr_ref[...], preferred_element_type=jnp.float32) + br_ref[...]
    xla_ref[...] = xl[:, :f]
    xlb_ref[...] = xl[:, f:]
    xra_ref[...] = xr[:, :f]
    xrb_ref[...] = xr[:, f:]


def _mm_halves(x, wl, wr, bl, br, fh):
    k = x.shape[1]
    f2 = 2 * fh
    rb = 1024
    out = jax.ShapeDtypeStruct((NP, fh), jnp.float32)
    return pl.pallas_call(
        _mm_kernel,
        grid=(NP // rb,),
        in_specs=[
            pl.BlockSpec((rb, k), lambda i: (i, 0)),
            pl.BlockSpec((k, f2), lambda i: (0, 0)),
            pl.BlockSpec((k, f2), lambda i: (0, 0)),
            pl.BlockSpec((1, f2), lambda i: (0, 0)),
            pl.BlockSpec((1, f2), lambda i: (0, 0)),
        ],
        out_specs=[pl.BlockSpec((rb, fh), lambda i: (i, 0))] * 4,
        out_shape=[out] * 4,
    )(x, wl, wr, bl.reshape(1, -1), br.reshape(1, -1))


def _mm2_kernel(ha_ref, hb_ref, wl_ref, wr_ref, bl_ref, br_ref,
                xla_ref, xlb_ref, xra_ref, xrb_ref):
    ha = ha_ref[...]
    hb = hb_ref[...]
    f = xla_ref.shape[1]
    xl = (jnp.dot(ha, wl_ref[:128, :], preferred_element_type=jnp.float32)
          + jnp.dot(hb, wl_ref[128:, :], preferred_element_type=jnp.float32)
          + bl_ref[...])
    xr = (jnp.dot(ha, wr_ref[:128, :], preferred_element_type=jnp.float32)
          + jnp.dot(hb, wr_ref[128:, :], preferred_element_type=jnp.float32)
          + br_ref[...])
    xla_ref[...] = xl[:, :f]
    xlb_ref[...] = xl[:, f:]
    xra_ref[...] = xr[:, :f]
    xrb_ref[...] = xr[:, f:]


def _mm2_halves(ha, hb, wl, wr, bl, br, fh):
    f2 = 2 * fh
    rb = 1024
    out = jax.ShapeDtypeStruct((NP, fh), jnp.float32)
    return pl.pallas_call(
        _mm2_kernel,
        grid=(NP // rb,),
        in_specs=[
            pl.BlockSpec((rb, 128), lambda i: (i, 0)),
            pl.BlockSpec((rb, 128), lambda i: (i, 0)),
            pl.BlockSpec((256, f2), lambda i: (0, 0)),
            pl.BlockSpec((256, f2), lambda i: (0, 0)),
            pl.BlockSpec((1, f2), lambda i: (0, 0)),
            pl.BlockSpec((1, f2), lambda i: (0, 0)),
        ],
        out_specs=[pl.BlockSpec((rb, fh), lambda i: (i, 0))] * 4,
        out_shape=[out] * 4,
    )(ha, hb, wl, wr, bl.reshape(1, -1), br.reshape(1, -1))


def _make_score_kernel(f, h):
    fh = f // 2
    c_per_h = f // h
    per_w = EP // 32
    n_chunks = per_w // CHUNK
    mesh = plsc.VectorSubcoreMesh(core_axis_name="c", subcore_axis_name="s")

    row_t = pltpu.VMEM((CHUNK, fh), jnp.float32)

    @functools.partial(
        pl.kernel,
        out_type=jax.ShapeDtypeStruct((h, EP), jnp.float32),
        mesh=mesh,
        compiler_params=pltpu.CompilerParams(use_tc_tiling_on_sc=False),
        scratch_types=[
            pltpu.VMEM((CHUNK,), jnp.int32),
            pltpu.VMEM((CHUNK,), jnp.int32),
            pltpu.VMEM((CHUNK,), jnp.int32),
            pltpu.VMEM((CHUNK,), jnp.int32),
            row_t, row_t, row_t, row_t,
            row_t, row_t, row_t, row_t,
            pltpu.VMEM((f,), jnp.float32),
            pltpu.VMEM((h, CHUNK), jnp.float32),
            pltpu.SemaphoreType.DMA,
            pltpu.SemaphoreType.DMA,
        ],
    )
    def score_k(xla, xlb, xra, xrb, src, dst, att, scores,
                is0, id0, is1, id1,
                rla0, rlb0, rra0, rrb0, rla1, rlb1, rra1, rrb1,
                att_v, sc_buf, sem0, sem1):
        cid = lax.axis_index("c")
        sid = lax.axis_index("s")
        wid = sid * 2 + cid
        base = wid * per_w
        pltpu.sync_copy(att, att_v)
        iota = lax.iota(jnp.int32, 16)
        bfly = [iota ^ k for k in (8, 4, 2, 1)]
        bufs = ((is0, id0, rla0, rlb0, rra0, rrb0, sem0),
                (is1, id1, rla1, rlb1, rra1, rrb1, sem1))

        def fetch(ci, buf):
            idx_s, idx_d, rla, rlb, rra, rrb, sem = buf
            start = base + ci * CHUNK
            pltpu.sync_copy(src.at[pl.ds(start, CHUNK)], idx_s)
            pltpu.sync_copy(dst.at[pl.ds(start, CHUNK)], idx_d)
            pltpu.async_copy(xla.at[idx_s], rla, sem)
            pltpu.async_copy(xlb.at[idx_s], rlb, sem)
            pltpu.async_copy(xra.at[idx_d], rra, sem)
            pltpu.async_copy(xrb.at[idx_d], rrb, sem)

        def sweep(ci, buf):
            idx_s, idx_d, rla, rlb, rra, rrb, sem = buf
            start = base + ci * CHUNK
            pltpu.make_async_copy(xla.at[idx_s], rla, sem).wait()
            pltpu.make_async_copy(xlb.at[idx_s], rlb, sem).wait()
            pltpu.make_async_copy(xra.at[idx_d], rra, sem).wait()
            pltpu.make_async_copy(xrb.at[idx_d], rrb, sem).wait()

            def grp_body(g, _):
                vecs = [jnp.zeros((16,), jnp.float32) for _ in range(h)]
                for j16 in range(16):
                    j = g * 16 + j16
                    for hh in range(h):
                        acc = jnp.zeros((16,), jnp.float32)
                        for qq in range(c_per_h // 16):
                            gg = hh * c_per_h + qq * 16
                            if gg < fh:
                                vl = rla[j, pl.ds(gg, 16)]
                                vr = rra[j, pl.ds(gg, 16)]
                            else:
                                vl = rlb[j, pl.ds(gg - fh, 16)]
                                vr = rrb[j, pl.ds(gg - fh, 16)]
                            v = vl + vr
                            lr = jnp.maximum(v, NEG_SLOPE * v)
                            acc = acc + lr * att_v[pl.ds(gg, 16)]
                        for ix in bfly:
                            acc = acc + _lane_shuffle(acc, ix)
                        vecs[hh] = jnp.where(iota == j16, acc, vecs[hh])
                for hh in range(h):
                    sc_buf[hh, pl.ds(g * 16, 16)] = vecs[hh]
                return 0

            lax.fori_loop(0, CHUNK // 16, grp_body, 0)

            @pl.when(ci + 2 < n_chunks)
            def _():
                fetch(ci + 2, buf)

            pltpu.sync_copy(sc_buf, scores.at[:, pl.ds(start, CHUNK)])

        fetch(0, bufs[0])
        fetch(1, bufs[1])

        def pair_body(g, _):
            sweep(2 * g, bufs[0])
            sweep(2 * g + 1, bufs[1])
            return 0

        lax.fori_loop(0, n_chunks // 2, pair_body, 0)

    return score_k


def _make_agg_kernel(f, h, relu, achunk):
    fh = f // 2
    heads_own = max(h // 2, 1)
    sl_per_head = (fh // 16) // heads_own
    w = fh + 16
    per_t = EP // 16
    n_chunks = per_t // achunk
    mesh = plsc.VectorSubcoreMesh(core_axis_name="c", subcore_axis_name="s")
    out = jax.ShapeDtypeStruct((NP, fh), jnp.float32)

    @functools.partial(
        pl.kernel,
        out_type=[out, out],
        mesh=mesh,
        compiler_params=pltpu.CompilerParams(use_tc_tiling_on_sc=False),
        scratch_types=[
            pltpu.VMEM((achunk,), jnp.int32),
            pltpu.VMEM((achunk,), jnp.int32),
            pltpu.VMEM((achunk,), jnp.int32),
            pltpu.VMEM((achunk,), jnp.int32),
            pltpu.VMEM((achunk, fh), jnp.float32),
            pltpu.VMEM((achunk, fh), jnp.float32),
            pltpu.VMEM((h, achunk), jnp.float32),
            pltpu.VMEM((achunk, w), jnp.float32),
            pltpu.VMEM((achunk, w), jnp.float32),
            pltpu.VMEM((16, w), jnp.float32),
            pltpu.VMEM((16, w), jnp.float32),
            pltpu.VMEM((16, fh), jnp.float32),
            pltpu.VMEM((fh,), jnp.float32),
            pltpu.VMEM_SHARED((ACC_ROWS, w), jnp.float32),
            pltpu.SemaphoreType.DMA,
            pltpu.SemaphoreType.DMA,
            pltpu.SemaphoreType.DMA,
            pltpu.SemaphoreType.DMA,
        ],
    )
    def agg_k(xla, xlb, src, dst, scores, bias, out_a, out_b,
              is0, is1, id0, id1, rows0, rows1, sc_chunk, stage0, stage1,
              zbuf, nblk, oblk, bias_v, acc, sem0, sem1, ssem0, ssem1):
        cid = lax.axis_index("c")
        sid = lax.axis_index("s")
        pltpu.sync_copy(bias.at[pl.ds(cid * fh, fh)], bias_v)

        zero = jnp.zeros((16,), jnp.float32)
        for r in range(16):
            for q in range(w // 16):
                zbuf[r, pl.ds(q * 16, 16)] = zero
        zblocks = ACC_ROWS // 16

        def zero_body(k, _):
            b = sid + 16 * k

            @pl.when(b < zblocks)
            def _():
                pltpu.sync_copy(zbuf, acc.at[pl.ds(b * 16, 16), :])

            return 0

        lax.fori_loop(0, (zblocks + 15) // 16, zero_body, 0)
        plsc.subcore_barrier()

        base = sid * per_t
        col0 = cid * (h // 2)
        iota = lax.iota(jnp.int32, 16)
        bufs = ((is0, id0, rows0, stage0, sem0, ssem0),
                (is1, id1, rows1, stage1, sem1, ssem1))

        def fetch(ci, buf):
            idx_s, idx_d, rows, stage, sem, ssem = buf
            start = base + ci * achunk
            pltpu.sync_copy(src.at[pl.ds(start, achunk)], idx_s)

            @pl.when(cid == 0)
            def _():
                pltpu.async_copy(xla.at[idx_s], rows, sem)

            @pl.when(cid == 1)
            def _():
                pltpu.async_copy(xlb.at[idx_s], rows, sem)

        def sweep(ci, buf, drain):
            idx_s, idx_d, rows, stage, sem, ssem = buf
            start = base + ci * achunk
            if drain:
                pltpu.make_async_copy(stage, acc.at[idx_d], ssem).wait()
            pltpu.make_async_copy(xla.at[idx_s], rows, sem).wait()
            pltpu.sync_copy(dst.at[pl.ds(start, achunk)], idx_d)
            pltpu.sync_copy(scores.at[:, pl.ds(start, achunk)], sc_chunk)

            def grp_body(g, _):
                ev_grp = []
                for hl in range(heads_own):
                    sv = sc_chunk[col0 + hl, pl.ds(g * 16, 16)]
                    ev_grp.append(jnp.exp(sv))
                for j16 in range(16):
                    j = g * 16 + j16
                    evecs = [jnp.full((16,), ev_grp[hl][j16], jnp.float32)
                             for hl in range(heads_own)]
                    for q in range(fh // 16):
                        hl = q // sl_per_head
                        sl = pl.ds(q * 16, 16)
                        stage[j, sl] = rows[j, sl] * evecs[hl]
                    tail = jnp.zeros((16,), jnp.float32)
                    for hl in range(heads_own):
                        tail = jnp.where(iota == hl, evecs[hl], tail)
                    stage[j, pl.ds(fh, 16)] = tail
                return 0

            lax.fori_loop(0, achunk // 16, grp_body, 0)

            @pl.when(ci + 2 < n_chunks)
            def _():
                fetch(ci + 2, buf)

            pltpu.async_copy(stage, acc.at[idx_d], ssem, add=True)

        fetch(0, bufs[0])
        fetch(1, bufs[1])
        sweep(0, bufs[0], False)
        sweep(1, bufs[1], False)

        def pair_body(g, _):
            sweep(2 * g, bufs[0], True)
            sweep(2 * g + 1, bufs[1], True)
            return 0

        lax.fori_loop(1, n_chunks // 2, pair_body, 0)
        for buf in bufs:
            idx_s, idx_d, rows, stage, sem, ssem = buf
            pltpu.make_async_copy(stage, acc.at[idx_d], ssem).wait()
        plsc.subcore_barrier()

        n_blocks = N // 16

        def norm_body(k, _):
            b = sid + 16 * k

            @pl.when(b < n_blocks)
            def _():
                r0 = b * 16
                pltpu.sync_copy(acc.at[pl.ds(r0, 16), :], nblk)
                for r in range(16):
                    dv = nblk[r, pl.ds(fh, 16)]
                    invs = []
                    for hl in range(heads_own):
                        invs.append(1.0 / (jnp.full((16,), dv[hl], jnp.float32) + 1e-16))
                    for q in range(fh // 16):
                        hl = q // sl_per_head
                        sl = pl.ds(q * 16, 16)
                        o = nblk[r, sl] * invs[hl] + bias_v[sl]
                        if relu:
                            o = jnp.maximum(o, 0.0)
                        oblk[r, sl] = o

                @pl.when(cid == 0)
                def _():
                    pltpu.sync_copy(oblk, out_a.at[pl.ds(r0, 16), :])

                @pl.when(cid == 1)
                def _():
                    pltpu.sync_copy(oblk, out_b.at[pl.ds(r0, 16), :])

            return 0

        lax.fori_loop(0, (n_blocks + 15) // 16, norm_body, 0)

    return agg_k


_score1 = _make_score_kernel(256, 4)
_agg1 = _make_agg_kernel(256, 4, True, 48)
_score2 = _make_score_kernel(128, 1)
_agg2 = _make_agg_kernel(128, 1, False, 128)


def kernel(x, edge_index, Wl1, bl1, Wr1, br1, att1, bias1,
           Wl2, bl2, Wr2, br2, att2, bias2):
    loop = jnp.arange(N, dtype=jnp.int32)
    padn = jnp.full((EP - E_REAL,), N, dtype=jnp.int32)
    src = jnp.concatenate([edge_index[0].astype(jnp.int32), loop, padn])
    dst = jnp.concatenate([edge_index[1].astype(jnp.int32), loop, padn])

    x_pad = jnp.zeros((NP, x.shape[1]), jnp.float32).at[:N].set(x)

    xla1, xlb1, xra1, xrb1 = _mm_halves(x_pad, Wl1, Wr1, bl1, br1, 128)
    s1 = _score1(xla1, xlb1, xra1, xrb1, src, dst, att1.reshape(-1))
    ha, hb = _agg1(xla1, xlb1, src, dst, s1, bias1)

    xla2, xlb2, xra2, xrb2 = _mm2_halves(ha, hb, Wl2, Wr2, bl2, br2, 64)
    s2 = _score2(xla2, xlb2, xra2, xrb2, src, dst, att2.reshape(-1))
    oa, ob = _agg2(xla2, xlb2, src, dst, s2, bias2)

    return jnp.concatenate([oa[:N], ob[:N]], axis=1)
